# baseline (speedup 1.0000x reference)
import functools

import jax
import jax.numpy as jnp
from jax import lax
from jax.experimental import pallas as pl
from jax.experimental.pallas import tpu as pltpu
from jax.experimental.pallas import tpu_sc as plsc

N = 10000
D = 128
DH = D // 2
E = 320000
NC = 2
NS = 16
EPS = E // NS
C = 125
NCHUNK = EPS // C
NCHD = NCHUNK // NC
RPT = 624
TAIL = N - NS * RPT
ZB = 48
DW = 16


def _zero_fill(buf, nrows, width):
    @pl.loop(0, nrows)
    def _(r):
        @pl.loop(0, width, step=16)
        def _(j):
            buf[r, pl.ds(j, 16)] = jnp.zeros((16,), jnp.float32)


def _zero_accum(sid, zb, accum, sem):
    @pl.loop(0, RPT // ZB)
    def _(k):
        pltpu.async_copy(zb, accum.at[pl.ds(sid * RPT + k * ZB, ZB)], sem)

    @pl.when(sid == NS - 1)
    def _():
        pltpu.async_copy(zb.at[pl.ds(0, TAIL)],
                         accum.at[pl.ds(NS * RPT, TAIL)], sem)

    @pl.loop(0, RPT // ZB)
    def _(k):
        pltpu.make_async_copy(
            zb, accum.at[pl.ds(sid * RPT + k * ZB, ZB)], sem).wait()

    @pl.when(sid == NS - 1)
    def _():
        pltpu.make_async_copy(
            zb.at[pl.ds(0, TAIL)], accum.at[pl.ds(NS * RPT, TAIL)],
            sem).wait()


def _drain_accum(cid, sid, accum, out_hbm):
    pltpu.sync_copy(accum.at[pl.ds(sid * RPT, RPT)],
                    out_hbm.at[cid, pl.ds(sid * RPT, RPT)])

    @pl.when(sid == NS - 1)
    def _():
        pltpu.sync_copy(accum.at[pl.ds(NS * RPT, TAIL)],
                        out_hbm.at[cid, pl.ds(NS * RPT, TAIL)])


def _sc_degree(dstw):
    mesh = plsc.VectorSubcoreMesh(core_axis_name="c", subcore_axis_name="s")

    @functools.partial(
        pl.kernel,
        out_type=jax.ShapeDtypeStruct((NC, N, DH), jnp.float32),
        mesh=mesh,
        scratch_types=[
            pltpu.VMEM((NCHD, C), jnp.int32),
            pltpu.VMEM((C, DW), jnp.float32),
            pltpu.VMEM((ZB, DW), jnp.float32),
            pltpu.VMEM((RPT + TAIL, DW), jnp.float32),
            pltpu.VMEM((RPT + TAIL, DH), jnp.float32),
            pltpu.VMEM_SHARED((N, DW), jnp.float32),
            pltpu.SemaphoreType.DMA,
            pltpu.SemaphoreType.DMA,
            pltpu.SemaphoreType.DMA,
        ],
        compiler_params=pltpu.CompilerParams(use_tc_tiling_on_sc=False),
    )
    def deg_kernel(edge_hbm, out_hbm, idxb, onesb, zb, tmp16, tmp64, accum,
                   sem_z, sem_i, sem_s):
        cid = lax.axis_index("c")
        sid = lax.axis_index("s")
        idx_src = edge_hbm.at[1, sid, pl.ds(cid * NCHD, NCHD)]
        pltpu.async_copy(idx_src, idxb, sem_i)
        _zero_fill(zb, ZB, DW)

        @pl.loop(0, C)
        def _(r):
            onesb[r, :] = jnp.ones((16,), jnp.float32)

        _zero_accum(sid, zb, accum, sem_z)
        plsc.subcore_barrier()
        pltpu.make_async_copy(idx_src, idxb, sem_i).wait()

        KF = 16

        @pl.loop(0, NCHD // KF)
        def _(g):
            @pl.loop(0, KF)
            def _(j):
                pltpu.async_copy(onesb, accum.at[idxb.at[g * KF + j]],
                                 sem_s, add=True)

            @pl.loop(0, KF)
            def _(j):
                pltpu.make_async_copy(onesb, accum.at[idxb.at[g * KF + j]],
                                      sem_s).wait()

        plsc.subcore_barrier()
        nmine = RPT + TAIL
        pltpu.sync_copy(accum.at[pl.ds(sid * RPT, RPT)],
                        tmp16.at[pl.ds(0, RPT)])

        @pl.when(sid == NS - 1)
        def _():
            pltpu.sync_copy(accum.at[pl.ds(NS * RPT, TAIL)],
                            tmp16.at[pl.ds(RPT, TAIL)])

        @pl.loop(0, nmine)
        def _(r):
            v = tmp16[r, :]
            tmp64[r, pl.ds(0, 16)] = v
            tmp64[r, pl.ds(16, 16)] = v
            tmp64[r, pl.ds(32, 16)] = v
            tmp64[r, pl.ds(48, 16)] = v

        pltpu.sync_copy(tmp64.at[pl.ds(0, RPT)],
                        out_hbm.at[cid, pl.ds(sid * RPT, RPT)])

        @pl.when(sid == NS - 1)
        def _():
            pltpu.sync_copy(tmp64.at[pl.ds(RPT, TAIL)],
                            out_hbm.at[cid, pl.ds(NS * RPT, TAIL)])

    return deg_kernel(dstw)


def _sc_message(values, eiw):
    mesh = plsc.VectorSubcoreMesh(core_axis_name="c", subcore_axis_name="s")

    @functools.partial(
        pl.kernel,
        out_type=jax.ShapeDtypeStruct((NC, N, DH), jnp.float32),
        mesh=mesh,
        scratch_types=[
            pltpu.VMEM((NCHUNK, C), jnp.int32),
            pltpu.VMEM((NCHUNK, C), jnp.int32),
            pltpu.VMEM((C, DH), jnp.float32),
            pltpu.VMEM((C, DH), jnp.float32),
            pltpu.VMEM((C, DH), jnp.float32),
            pltpu.VMEM((C, DH), jnp.float32),
            pltpu.VMEM((ZB, DH), jnp.float32),
            pltpu.VMEM_SHARED((N, DH), jnp.float32),
            pltpu.SemaphoreType.DMA,
            pltpu.SemaphoreType.DMA,
            pltpu.SemaphoreType.DMA,
            pltpu.SemaphoreType.DMA,
            pltpu.SemaphoreType.DMA,
            pltpu.SemaphoreType.DMA,
            pltpu.SemaphoreType.DMA,
            pltpu.SemaphoreType.DMA,
            pltpu.SemaphoreType.DMA,
            pltpu.SemaphoreType.DMA,
        ],
        compiler_params=pltpu.CompilerParams(use_tc_tiling_on_sc=False),
    )
    def msg_kernel(val_hbm, edge_hbm, out_hbm, srcb, dstb, rows0,
                   rows1, rows2, rows3, zb, accum, sem_z, sem_i, sg0, sg1,
                   sg2, sg3, ss0, ss1, ss2, ss3):
        cid = lax.axis_index("c")
        sid = lax.axis_index("s")
        vals = val_hbm.at[cid]
        bufs = (rows0, rows1, rows2, rows3)
        sems = (sg0, sg1, sg2, sg3)
        ssems = (ss0, ss1, ss2, ss3)
        pltpu.async_copy(edge_hbm.at[0, sid], srcb, sem_i)
        pltpu.async_copy(edge_hbm.at[1, sid], dstb, sem_i)
        _zero_fill(zb, ZB, DH)
        _zero_accum(sid, zb, accum, sem_z)
        plsc.subcore_barrier()
        pltpu.make_async_copy(edge_hbm.at[0, sid], srcb, sem_i).wait()
        pltpu.make_async_copy(edge_hbm.at[1, sid], dstb, sem_i).wait()

        def gather(i, j):
            pltpu.async_copy(vals.at[srcb.at[i]], bufs[j], sems[j])

        def wait_g(i, j):
            pltpu.make_async_copy(vals.at[srcb.at[i]], bufs[j], sems[j]).wait()

        def scatter(i, j):
            pltpu.sync_copy(bufs[j], accum.at[dstb.at[i]], add=True)

        gather(0, 0)
        gather(1, 1)
        gather(2, 2)

        @pl.loop(0, NCHUNK // 4 - 1)
        def _(g):
            i0 = 4 * g
            for j in range(4):
                i = i0 + j
                gather(i + 3, (j + 3) % 4)
                wait_g(i, j)
                scatter(i, j)

        base = NCHUNK - 4
        gather(NCHUNK - 1, 3)
        for j in range(4):
            wait_g(base + j, j)
            scatter(base + j, j)

        plsc.subcore_barrier()
        _drain_accum(cid, sid, accum, out_hbm)

    return msg_kernel(values, eiw)


NP = N // 2
_BR = 1000


def _bvec(b_ref, c):
    half = b_ref[:, DH * c:DH * (c + 1)]
    return jnp.concatenate([half, half], axis=1)


def _dis(degb_ref):
    return lax.rsqrt(1.0 + degb_ref[0] + degb_ref[1])


def _tc_hws1(degb, x, W_in, b_in, W1):
    def body(degb_ref, x_ref, win_ref, bin_ref, w1_ref, o_ref):
        dis = _dis(degb_ref)
        wc = jnp.dot(win_ref[...], w1_ref[...],
                     preferred_element_type=jnp.float32)
        bc = jnp.dot(bin_ref[...], w1_ref[...],
                     preferred_element_type=jnp.float32)
        hw = jnp.dot(x_ref[...], wc, preferred_element_type=jnp.float32) + bc
        he = hw.reshape(_BR, 2, D)[:, 0, :]
        ho = hw.reshape(_BR, 2, D)[:, 1, :]
        for d in range(NC):
            o_ref[d] = jnp.concatenate(
                [he[:, DH * d:DH * (d + 1)], ho[:, DH * d:DH * (d + 1)]],
                axis=1) * dis

    return pl.pallas_call(
        body,
        grid=(NP // _BR,),
        in_specs=[
            pl.BlockSpec((NC, _BR, D), lambda i: (0, i, 0)),
            pl.BlockSpec((2 * _BR, D), lambda i: (i, 0)),
            pl.BlockSpec((D, D), lambda i: (0, 0)),
            pl.BlockSpec((1, D), lambda i: (0, 0)),
            pl.BlockSpec((D, D), lambda i: (0, 0)),
        ],
        out_specs=pl.BlockSpec((NC, _BR, D), lambda i: (0, i, 0)),
        out_shape=jax.ShapeDtypeStruct((NC, NP, D), jnp.float32),
    )(degb, x, W_in, b_in.reshape(1, D), W1)


def _tc_mid(degb, mp, hws1, b1, W2):
    def body(degb_ref, mp_ref, hws1_ref, b1_ref, w2_ref, o_ref):
        dis = _dis(degb_ref)
        h1 = []
        for c in range(NC):
            s = mp_ref[c] + hws1_ref[c]
            h1.append(jnp.maximum(dis * s + _bvec(b1_ref, c), 0.0))
        even = jnp.concatenate([h1[0][:, :DH], h1[1][:, :DH]], axis=1)
        odd = jnp.concatenate([h1[0][:, DH:], h1[1][:, DH:]], axis=1)
        h1n = jnp.stack([even, odd], axis=1).reshape(2 * _BR, D)
        hw2 = jnp.dot(h1n, w2_ref[...], preferred_element_type=jnp.float32)
        he = hw2.reshape(_BR, 2, D)[:, 0, :]
        ho = hw2.reshape(_BR, 2, D)[:, 1, :]
        for d in range(NC):
            o_ref[d] = jnp.concatenate(
                [he[:, DH * d:DH * (d + 1)], ho[:, DH * d:DH * (d + 1)]],
                axis=1) * dis

    return pl.pallas_call(
        body,
        grid=(NP // _BR,),
        in_specs=[
            pl.BlockSpec((NC, _BR, D), lambda i: (0, i, 0)),
            pl.BlockSpec((NC, _BR, D), lambda i: (0, i, 0)),
            pl.BlockSpec((NC, _BR, D), lambda i: (0, i, 0)),
            pl.BlockSpec((1, D), lambda i: (0, 0)),
            pl.BlockSpec((D, D), lambda i: (0, 0)),
        ],
        out_specs=pl.BlockSpec((NC, _BR, D), lambda i: (0, i, 0)),
        out_shape=jax.ShapeDtypeStruct((NC, NP, D), jnp.float32),
    )(degb, mp, hws1, b1.reshape(1, D), W2)


def _tc_final(degb, mp, hws2, b2):
    def body(degb_ref, mp_ref, hws2_ref, b2_ref, o_ref):
        dis = _dis(degb_ref)
        o = []
        for c in range(NC):
            s = mp_ref[c] + hws2_ref[c]
            o.append(dis * s + _bvec(b2_ref, c))
        even = jnp.concatenate([o[0][:, :DH], o[1][:, :DH]], axis=1)
        odd = jnp.concatenate([o[0][:, DH:], o[1][:, DH:]], axis=1)
        o_ref[...] = jnp.stack([even, odd], axis=1).reshape(2 * _BR, D)

    return pl.pallas_call(
        body,
        grid=(NP // _BR,),
        in_specs=[
            pl.BlockSpec((NC, _BR, D), lambda i: (0, i, 0)),
            pl.BlockSpec((NC, _BR, D), lambda i: (0, i, 0)),
            pl.BlockSpec((NC, _BR, D), lambda i: (0, i, 0)),
            pl.BlockSpec((1, D), lambda i: (0, 0)),
        ],
        out_specs=pl.BlockSpec((2 * _BR, D), lambda i: (i, 0)),
        out_shape=jax.ShapeDtypeStruct((N, D), jnp.float32),
    )(degb, mp, hws2, b2.reshape(1, D))


def kernel(x, edge_index, W_in, b_in, W1, b1, W2, b2):
    eiw = edge_index.astype(jnp.int32).reshape(2, NS, NCHUNK, C)
    degb = _sc_degree(eiw).reshape(NC, NP, D)
    hws1 = _tc_hws1(degb, x, W_in, b_in, W1)
    mp1 = _sc_message(hws1.reshape(NC, N, DH), eiw)
    hws2 = _tc_mid(degb, mp1.reshape(NC, NP, D), hws1, b1, W2)
    mp2 = _sc_message(hws2.reshape(NC, N, DH), eiw)
    return _tc_final(degb, mp2.reshape(NC, NP, D), hws2, b2)


# speedup vs baseline: 1.0165x; 1.0165x over previous
import functools

import jax
import jax.numpy as jnp
from jax import lax
from jax.experimental import pallas as pl
from jax.experimental.pallas import tpu as pltpu
from jax.experimental.pallas import tpu_sc as plsc

N = 10000
D = 128
DH = D // 2
E = 320000
NC = 2
NS = 16
EPS = E // NS
C = 125
NCHUNK = EPS // C
NCHD = NCHUNK // NC
RPT = 624
TAIL = N - NS * RPT
ZB = 48
DW = 16


def _zero_fill(buf, nrows, width):
    @pl.loop(0, nrows)
    def _(r):
        @pl.loop(0, width, step=16)
        def _(j):
            buf[r, pl.ds(j, 16)] = jnp.zeros((16,), jnp.float32)


def _zero_accum(sid, zb, accum, sem):
    @pl.loop(0, RPT // ZB)
    def _(k):
        pltpu.async_copy(zb, accum.at[pl.ds(sid * RPT + k * ZB, ZB)], sem)

    @pl.when(sid == NS - 1)
    def _():
        pltpu.async_copy(zb.at[pl.ds(0, TAIL)],
                         accum.at[pl.ds(NS * RPT, TAIL)], sem)

    @pl.loop(0, RPT // ZB)
    def _(k):
        pltpu.make_async_copy(
            zb, accum.at[pl.ds(sid * RPT + k * ZB, ZB)], sem).wait()

    @pl.when(sid == NS - 1)
    def _():
        pltpu.make_async_copy(
            zb.at[pl.ds(0, TAIL)], accum.at[pl.ds(NS * RPT, TAIL)],
            sem).wait()


def _drain_accum(cid, sid, accum, out_hbm):
    pltpu.sync_copy(accum.at[pl.ds(sid * RPT, RPT)],
                    out_hbm.at[cid, pl.ds(sid * RPT, RPT)])

    @pl.when(sid == NS - 1)
    def _():
        pltpu.sync_copy(accum.at[pl.ds(NS * RPT, TAIL)],
                        out_hbm.at[cid, pl.ds(NS * RPT, TAIL)])


def _sc_degree(dstw):
    mesh = plsc.VectorSubcoreMesh(core_axis_name="c", subcore_axis_name="s")

    @functools.partial(
        pl.kernel,
        out_type=jax.ShapeDtypeStruct((NC, N, DH), jnp.float32),
        mesh=mesh,
        scratch_types=[
            pltpu.VMEM((NCHD, C), jnp.int32),
            pltpu.VMEM((C, DW), jnp.float32),
            pltpu.VMEM((ZB, DW), jnp.float32),
            pltpu.VMEM((RPT + TAIL, DW), jnp.float32),
            pltpu.VMEM((RPT + TAIL, DH), jnp.float32),
            pltpu.VMEM_SHARED((N, DW), jnp.float32),
            pltpu.SemaphoreType.DMA,
            pltpu.SemaphoreType.DMA,
            pltpu.SemaphoreType.DMA,
        ],
        compiler_params=pltpu.CompilerParams(use_tc_tiling_on_sc=False),
    )
    def deg_kernel(edge_hbm, out_hbm, idxb, onesb, zb, tmp16, tmp64, accum,
                   sem_z, sem_i, sem_s):
        cid = lax.axis_index("c")
        sid = lax.axis_index("s")
        idx_src = edge_hbm.at[1, sid, pl.ds(cid * NCHD, NCHD)]
        pltpu.async_copy(idx_src, idxb, sem_i)
        _zero_fill(zb, ZB, DW)

        @pl.loop(0, C)
        def _(r):
            onesb[r, :] = jnp.ones((16,), jnp.float32)

        _zero_accum(sid, zb, accum, sem_z)
        plsc.subcore_barrier()
        pltpu.make_async_copy(idx_src, idxb, sem_i).wait()

        KF = 16

        @pl.loop(0, NCHD // KF)
        def _(g):
            @pl.loop(0, KF)
            def _(j):
                pltpu.async_copy(onesb, accum.at[idxb.at[g * KF + j]],
                                 sem_s, add=True)

            @pl.loop(0, KF)
            def _(j):
                pltpu.make_async_copy(onesb, accum.at[idxb.at[g * KF + j]],
                                      sem_s).wait()

        plsc.subcore_barrier()
        nmine = RPT + TAIL
        pltpu.sync_copy(accum.at[pl.ds(sid * RPT, RPT)],
                        tmp16.at[pl.ds(0, RPT)])

        @pl.when(sid == NS - 1)
        def _():
            pltpu.sync_copy(accum.at[pl.ds(NS * RPT, TAIL)],
                            tmp16.at[pl.ds(RPT, TAIL)])

        @pl.loop(0, nmine)
        def _(r):
            v = tmp16[r, :]
            tmp64[r, pl.ds(0, 16)] = v
            tmp64[r, pl.ds(16, 16)] = v
            tmp64[r, pl.ds(32, 16)] = v
            tmp64[r, pl.ds(48, 16)] = v

        pltpu.sync_copy(tmp64.at[pl.ds(0, RPT)],
                        out_hbm.at[cid, pl.ds(sid * RPT, RPT)])

        @pl.when(sid == NS - 1)
        def _():
            pltpu.sync_copy(tmp64.at[pl.ds(RPT, TAIL)],
                            out_hbm.at[cid, pl.ds(NS * RPT, TAIL)])

    return deg_kernel(dstw)


def _sc_message(values, eiw):
    mesh = plsc.VectorSubcoreMesh(core_axis_name="c", subcore_axis_name="s")

    @functools.partial(
        pl.kernel,
        out_type=jax.ShapeDtypeStruct((NC, N, DH), jnp.float32),
        mesh=mesh,
        scratch_types=[
            pltpu.VMEM((NCHUNK, C), jnp.int32),
            pltpu.VMEM((NCHUNK, C), jnp.int32),
            pltpu.VMEM((C, DH), jnp.float32),
            pltpu.VMEM((C, DH), jnp.float32),
            pltpu.VMEM((C, DH), jnp.float32),
            pltpu.VMEM((C, DH), jnp.float32),
            pltpu.VMEM((ZB, DH), jnp.float32),
            pltpu.VMEM_SHARED((N, DH), jnp.float32),
            pltpu.SemaphoreType.DMA,
            pltpu.SemaphoreType.DMA,
            pltpu.SemaphoreType.DMA,
            pltpu.SemaphoreType.DMA,
            pltpu.SemaphoreType.DMA,
            pltpu.SemaphoreType.DMA,
            pltpu.SemaphoreType.DMA,
            pltpu.SemaphoreType.DMA,
            pltpu.SemaphoreType.DMA,
            pltpu.SemaphoreType.DMA,
        ],
        compiler_params=pltpu.CompilerParams(use_tc_tiling_on_sc=False),
    )
    def msg_kernel(val_hbm, edge_hbm, out_hbm, srcb, dstb, rows0,
                   rows1, rows2, rows3, zb, accum, sem_z, sem_i, sg0, sg1,
                   sg2, sg3, ss0, ss1, ss2, ss3):
        cid = lax.axis_index("c")
        sid = lax.axis_index("s")
        vals = val_hbm.at[cid]
        bufs = (rows0, rows1, rows2, rows3)
        sems = (sg0, sg1, sg2, sg3)
        ssems = (ss0, ss1, ss2, ss3)
        pltpu.async_copy(edge_hbm.at[0, sid], srcb, sem_i)
        pltpu.async_copy(edge_hbm.at[1, sid], dstb, sem_i)
        _zero_fill(zb, ZB, DH)
        _zero_accum(sid, zb, accum, sem_z)
        plsc.subcore_barrier()
        pltpu.make_async_copy(edge_hbm.at[0, sid], srcb, sem_i).wait()
        pltpu.make_async_copy(edge_hbm.at[1, sid], dstb, sem_i).wait()

        def gather(i, j):
            pltpu.async_copy(vals.at[srcb.at[i]], bufs[j], sems[j])

        def wait_g(i, j):
            pltpu.make_async_copy(vals.at[srcb.at[i]], bufs[j], sems[j]).wait()

        def scatter(i, j):
            pltpu.sync_copy(bufs[j], accum.at[dstb.at[i]], add=True)

        gather(0, 0)
        gather(1, 1)
        gather(2, 2)

        @pl.loop(0, NCHUNK // 4 - 1)
        def _(g):
            i0 = 4 * g
            for j in range(4):
                i = i0 + j
                gather(i + 3, (j + 3) % 4)
                wait_g(i, j)
                scatter(i, j)

        base = NCHUNK - 4
        gather(NCHUNK - 1, 3)
        for j in range(4):
            wait_g(base + j, j)
            scatter(base + j, j)

        plsc.subcore_barrier()
        _drain_accum(cid, sid, accum, out_hbm)

    return msg_kernel(values, eiw)


NP = N // 2
_BR = 1000


def _bd(w, c, d):
    wcd = w[DH * c:DH * (c + 1), DH * d:DH * (d + 1)]
    z = jnp.zeros((DH, DH), jnp.float32)
    return jnp.concatenate([jnp.concatenate([wcd, z], axis=1),
                            jnp.concatenate([z, wcd], axis=1)], axis=0)


def _bvec(b_ref, c):
    half = b_ref[:, DH * c:DH * (c + 1)]
    return jnp.concatenate([half, half], axis=1)


def _dis(degb_ref):
    return lax.rsqrt(1.0 + degb_ref[0] + degb_ref[1])


def _tc_hws1(degb, x, W_in, b_in, W1):
    def body(degb_ref, x_ref, win_ref, bin_ref, w1_ref, o_ref):
        dis = _dis(degb_ref)
        wc = jnp.dot(win_ref[...], w1_ref[...],
                     preferred_element_type=jnp.float32)
        bc = jnp.dot(bin_ref[...], w1_ref[...],
                     preferred_element_type=jnp.float32)
        hw = jnp.dot(x_ref[...], wc, preferred_element_type=jnp.float32) + bc
        he = hw.reshape(_BR, 2, D)[:, 0, :]
        ho = hw.reshape(_BR, 2, D)[:, 1, :]
        for d in range(NC):
            o_ref[d] = jnp.concatenate(
                [he[:, DH * d:DH * (d + 1)], ho[:, DH * d:DH * (d + 1)]],
                axis=1) * dis

    return pl.pallas_call(
        body,
        grid=(NP // _BR,),
        in_specs=[
            pl.BlockSpec((NC, _BR, D), lambda i: (0, i, 0)),
            pl.BlockSpec((2 * _BR, D), lambda i: (i, 0)),
            pl.BlockSpec((D, D), lambda i: (0, 0)),
            pl.BlockSpec((1, D), lambda i: (0, 0)),
            pl.BlockSpec((D, D), lambda i: (0, 0)),
        ],
        out_specs=pl.BlockSpec((NC, _BR, D), lambda i: (0, i, 0)),
        out_shape=jax.ShapeDtypeStruct((NC, NP, D), jnp.float32),
    )(degb, x, W_in, b_in.reshape(1, D), W1)


def _tc_mid(degb, mp, hws1, b1, W2):
    def body(degb_ref, mp_ref, hws1_ref, b1_ref, w2_ref, o_ref):
        dis = _dis(degb_ref)
        h1 = []
        for c in range(NC):
            s = mp_ref[c] + hws1_ref[c]
            h1.append(jnp.maximum(dis * s + _bvec(b1_ref, c), 0.0))
        for d in range(NC):
            acc = jnp.dot(h1[0], _bd(w2_ref[...], 0, d),
                          preferred_element_type=jnp.float32)
            acc = acc + jnp.dot(h1[1], _bd(w2_ref[...], 1, d),
                                preferred_element_type=jnp.float32)
            o_ref[d] = acc * dis

    return pl.pallas_call(
        body,
        grid=(NP // _BR,),
        in_specs=[
            pl.BlockSpec((NC, _BR, D), lambda i: (0, i, 0)),
            pl.BlockSpec((NC, _BR, D), lambda i: (0, i, 0)),
            pl.BlockSpec((NC, _BR, D), lambda i: (0, i, 0)),
            pl.BlockSpec((1, D), lambda i: (0, 0)),
            pl.BlockSpec((D, D), lambda i: (0, 0)),
        ],
        out_specs=pl.BlockSpec((NC, _BR, D), lambda i: (0, i, 0)),
        out_shape=jax.ShapeDtypeStruct((NC, NP, D), jnp.float32),
    )(degb, mp, hws1, b1.reshape(1, D), W2)


def _tc_final(degb, mp, hws2, b2):
    def body(degb_ref, mp_ref, hws2_ref, b2_ref, o_ref):
        dis = _dis(degb_ref)
        o = []
        for c in range(NC):
            s = mp_ref[c] + hws2_ref[c]
            o.append(dis * s + _bvec(b2_ref, c))
        even = jnp.concatenate([o[0][:, :DH], o[1][:, :DH]], axis=1)
        odd = jnp.concatenate([o[0][:, DH:], o[1][:, DH:]], axis=1)
        o_ref[...] = jnp.stack([even, odd], axis=1).reshape(2 * _BR, D)

    return pl.pallas_call(
        body,
        grid=(NP // _BR,),
        in_specs=[
            pl.BlockSpec((NC, _BR, D), lambda i: (0, i, 0)),
            pl.BlockSpec((NC, _BR, D), lambda i: (0, i, 0)),
            pl.BlockSpec((NC, _BR, D), lambda i: (0, i, 0)),
            pl.BlockSpec((1, D), lambda i: (0, 0)),
        ],
        out_specs=pl.BlockSpec((2 * _BR, D), lambda i: (i, 0)),
        out_shape=jax.ShapeDtypeStruct((N, D), jnp.float32),
    )(degb, mp, hws2, b2.reshape(1, D))


def kernel(x, edge_index, W_in, b_in, W1, b1, W2, b2):
    eiw = edge_index.astype(jnp.int32).reshape(2, NS, NCHUNK, C)
    degb = _sc_degree(eiw).reshape(NC, NP, D)
    hws1 = _tc_hws1(degb, x, W_in, b_in, W1)
    mp1 = _sc_message(hws1.reshape(NC, N, DH), eiw)
    hws2 = _tc_mid(degb, mp1.reshape(NC, NP, D), hws1, b1, W2)
    mp2 = _sc_message(hws2.reshape(NC, N, DH), eiw)
    return _tc_final(degb, mp2.reshape(NC, NP, D), hws2, b2)
